# Initial kernel scaffold; baseline (speedup 1.0000x reference)
import jax
import jax.numpy as jnp
from jax.experimental import pallas as pl


def kernel(x_num, x_cat, tables, W0, b0, g0, be0, W1, b1, g1, be1, W2, b2, g2, be2, W3, b3):
    raise NotImplementedError("write your pallas kernel here")


# speedup vs baseline: 7.4295x; 7.4295x over previous
import functools

import jax
import jax.numpy as jnp
from jax import lax
from jax.experimental import pallas as pl
from jax.experimental.pallas import tpu as pltpu
from jax.experimental.pallas import tpu_sc as plsc

N_FIELDS = 26
VOCAB = 100000
EMB = 16
EPS = 1e-5

NW = 32
CHUNK = 128


def _gather_kernel(rows, nch):
    rpw = rows // NW
    mesh = plsc.VectorSubcoreMesh(core_axis_name="c", subcore_axis_name="s")

    @functools.partial(
        pl.kernel,
        out_type=jax.ShapeDtypeStruct((rows, EMB), jnp.float32),
        mesh=mesh,
        compiler_params=pltpu.CompilerParams(use_tc_tiling_on_sc=False),
        scratch_types=[
            pltpu.VMEM((nch, CHUNK), jnp.int32),
            pltpu.VMEM((CHUNK, EMB), jnp.float32),
            pltpu.SemaphoreType.DMA,
        ],
    )
    def gk(tab_hbm, idx_hbm, out_hbm, idx_v, rows_v, gsem):
        wid = lax.axis_index("s") * 2 + lax.axis_index("c")
        base = wid * rpw
        pltpu.sync_copy(idx_hbm.at[wid], idx_v)

        def chunk_body(c, carry):
            pltpu.async_copy(tab_hbm.at[idx_v.at[c]], rows_v, gsem).wait()
            pltpu.sync_copy(rows_v, out_hbm.at[pl.ds(base + c * CHUNK, CHUNK)])
            return carry

        lax.fori_loop(0, nch, chunk_body, 0)

    return gk


def _mlp_body(xn_ref, emb_ref, w0n_ref, w0e_ref, b0_ref, w1_ref, b1_ref,
              w2_ref, b2_ref, w3_ref, b3_ref, out_ref):
    h = jnp.dot(emb_ref[...], w0e_ref[...], preferred_element_type=jnp.float32)
    h = h + jnp.dot(xn_ref[...], w0n_ref[...], preferred_element_type=jnp.float32)
    h = jnp.maximum(h + b0_ref[...], 0.0)
    h = jnp.maximum(jnp.dot(h, w1_ref[...], preferred_element_type=jnp.float32)
                    + b1_ref[...], 0.0)
    h = jnp.maximum(jnp.dot(h, w2_ref[...], preferred_element_type=jnp.float32)
                    + b2_ref[...], 0.0)
    out_ref[...] = jnp.dot(h, w3_ref[...], preferred_element_type=jnp.float32) + b3_ref[...]


def kernel(x_num, x_cat, tables, W0, b0, g0, be0, W1, b1, g1, be1,
           W2, b2, g2, be2, W3, b3):
    B = x_num.shape[0]
    rows = B * N_FIELDS
    rpw = rows // NW
    nch = rpw // CHUNK

    tab = tables.reshape(N_FIELDS * VOCAB, EMB)
    offs = (jnp.arange(N_FIELDS, dtype=jnp.int32) * VOCAB)[None, :]
    idx = (x_cat.astype(jnp.int32) + offs).reshape(NW, nch, CHUNK)

    inv = 1.0 / jnp.sqrt(1.0 + EPS)
    s0, s1, s2 = g0 * inv, g1 * inv, g2 * inv
    W0f = W0 * s0[None, :]
    b0f = (b0 * s0 + be0)[None, :]
    W1f = W1 * s1[None, :]
    b1f = (b1 * s1 + be1)[None, :]
    W2f = W2 * s2[None, :]
    b2f = (b2 * s2 + be2)[None, :]
    W0n = jnp.pad(W0f[:13], ((0, 3), (0, 0)))
    W0e = W0f[13:]
    w3p = jnp.pad(W3, ((0, 0), (0, 127)))
    b3p = jnp.pad(b3.reshape(1, 1), ((0, 0), (0, 127)))
    xn = jnp.pad(x_num, ((0, 0), (0, 3)))

    emb = _gather_kernel(rows, nch)(tab, idx)
    emb2 = emb.reshape(B, N_FIELDS * EMB)

    BM = 2048
    nb = B // BM
    full = lambda s: pl.BlockSpec(s, lambda i: (0, 0))
    out2 = pl.pallas_call(
        _mlp_body,
        grid=(nb,),
        in_specs=[
            pl.BlockSpec((BM, 16), lambda i: (i, 0)),
            pl.BlockSpec((BM, N_FIELDS * EMB), lambda i: (i, 0)),
            full((16, 128)), full((416, 128)), full((1, 128)),
            full((128, 64)), full((1, 64)),
            full((64, 32)), full((1, 32)),
            full((32, 128)), full((1, 128)),
        ],
        out_specs=pl.BlockSpec((BM, 128), lambda i: (i, 0)),
        out_shape=jax.ShapeDtypeStruct((B, 128), jnp.float32),
    )(xn, emb2, W0n, W0e, b0f, W1f, b1f, W2f, b2f, w3p, b3p)

    return out2[:, 0]
